# baseline (speedup 1.0000x reference)
import functools

import jax
import jax.numpy as jnp
from jax import lax
from jax.experimental import pallas as pl
from jax.experimental.pallas import tpu as pltpu
from jax.experimental.pallas import tpu_sc as plsc

D_MODEL = 1024
NUM_EXPERTS = 8
TOP_K = 2
N_TOKENS = 4096
N_PAIRS = N_TOKENS * TOP_K

TILE = 256
P_PAD = ((N_PAIRS + NUM_EXPERTS * (TILE - 1)) + TILE - 1) // TILE * TILE
NUM_TILES = P_PAD // TILE

_SC_INFO = plsc.get_sparse_core_info()
NUM_WORKERS = _SC_INFO.num_cores * _SC_INFO.num_subcores
TOK_PER_WORKER = N_TOKENS // NUM_WORKERS
CHUNK = 16


def _gate_kernel(x_ref, gw_ref, gb_ref, e1_ref, e2_ref, g1_ref, g2_ref):
    n = x_ref.shape[0]
    logits = jnp.dot(x_ref[...], gw_ref[...],
                     preferred_element_type=jnp.float32) + gb_ref[...]
    col = lax.broadcasted_iota(jnp.int32, (n, NUM_EXPERTS), 1)
    m1 = jnp.max(logits, axis=1, keepdims=True)
    a1 = jnp.min(jnp.where(logits == m1, col, NUM_EXPERTS), axis=1, keepdims=True)
    neg = jnp.float32(-jnp.inf)
    l2 = jnp.where(col == a1, neg, logits)
    m2 = jnp.max(l2, axis=1, keepdims=True)
    a2 = jnp.min(jnp.where(l2 == m2, col, NUM_EXPERTS), axis=1, keepdims=True)
    g1 = 1.0 / (1.0 + jnp.exp(m2 - m1))
    e1_ref[...] = a1
    e2_ref[...] = a2
    g1_ref[...] = g1
    g2_ref[...] = 1.0 - g1


def _gating(x, gate_W, gate_b):
    gb2 = gate_b.reshape(1, NUM_EXPERTS)
    grid = 16
    blk = N_TOKENS // grid
    outs = [
        jax.ShapeDtypeStruct((N_TOKENS, 1), jnp.int32),
        jax.ShapeDtypeStruct((N_TOKENS, 1), jnp.int32),
        jax.ShapeDtypeStruct((N_TOKENS, 1), jnp.float32),
        jax.ShapeDtypeStruct((N_TOKENS, 1), jnp.float32),
    ]
    ospec = pl.BlockSpec((blk, 1), lambda i: (i, 0))
    return pl.pallas_call(
        _gate_kernel,
        grid=(grid,),
        in_specs=[
            pl.BlockSpec((blk, D_MODEL), lambda i: (i, 0)),
            pl.BlockSpec((D_MODEL, NUM_EXPERTS), lambda i: (0, 0)),
            pl.BlockSpec((1, NUM_EXPERTS), lambda i: (0, 0)),
        ],
        out_specs=[ospec, ospec, ospec, ospec],
        out_shape=outs,
    )(x, gate_W, gb2)


def _make_scatter():
    mesh = plsc.VectorSubcoreMesh(core_axis_name="c", subcore_axis_name="s")

    @functools.partial(
        pl.kernel,
        mesh=mesh,
        out_type=jax.ShapeDtypeStruct((P_PAD, D_MODEL), jnp.float32),
        scratch_types=[
            pltpu.VMEM((CHUNK,), jnp.int32),
            pltpu.VMEM((CHUNK,), jnp.int32),
            pltpu.VMEM((CHUNK, D_MODEL), jnp.float32),
            pltpu.SemaphoreType.DMA,
            pltpu.SemaphoreType.DMA,
        ],
    )
    def scatter(x_hbm, d0_hbm, d1_hbm, xs_hbm, i0_v, i1_v, rows_v, sem0, sem1):
        wid = lax.axis_index("s") * _SC_INFO.num_cores + lax.axis_index("c")
        base = wid * TOK_PER_WORKER

        def body(c, carry):
            tb = base + c * CHUNK
            pltpu.sync_copy(d0_hbm.at[pl.ds(tb, CHUNK)], i0_v)
            pltpu.sync_copy(d1_hbm.at[pl.ds(tb, CHUNK)], i1_v)
            pltpu.sync_copy(x_hbm.at[pl.ds(tb, CHUNK)], rows_v)
            c0 = pltpu.async_copy(rows_v, xs_hbm.at[i0_v], sem0)
            c1 = pltpu.async_copy(rows_v, xs_hbm.at[i1_v], sem1)
            c0.wait()
            c1.wait()
            return carry

        lax.fori_loop(0, TOK_PER_WORKER // CHUNK, body, 0)

    return scatter


def _gmm_kernel(te_ref, xs_ref, w_ref, b_ref, y_ref):
    y_ref[...] = jnp.dot(xs_ref[...], w_ref[0],
                         preferred_element_type=jnp.float32) + b_ref[0]


def _gmm(tile_expert, xs, expert_W, expert_b):
    grid_spec = pltpu.PrefetchScalarGridSpec(
        num_scalar_prefetch=1,
        grid=(NUM_TILES,),
        in_specs=[
            pl.BlockSpec((TILE, D_MODEL), lambda i, te: (i, 0)),
            pl.BlockSpec((1, D_MODEL, D_MODEL), lambda i, te: (te[i], 0, 0)),
            pl.BlockSpec((1, 1, D_MODEL), lambda i, te: (te[i], 0, 0)),
        ],
        out_specs=pl.BlockSpec((TILE, D_MODEL), lambda i, te: (i, 0)),
    )
    return pl.pallas_call(
        _gmm_kernel,
        grid_spec=grid_spec,
        out_shape=jax.ShapeDtypeStruct((P_PAD, D_MODEL), jnp.float32),
        compiler_params=pltpu.CompilerParams(
            dimension_semantics=("arbitrary",)),
    )(tile_expert, xs, expert_W,
      expert_b.reshape(NUM_EXPERTS, 1, D_MODEL))


def _make_combine():
    mesh = plsc.VectorSubcoreMesh(core_axis_name="c", subcore_axis_name="s")

    @functools.partial(
        pl.kernel,
        mesh=mesh,
        out_type=jax.ShapeDtypeStruct((N_TOKENS, D_MODEL), jnp.float32),
        scratch_types=[
            pltpu.VMEM((CHUNK,), jnp.int32),
            pltpu.VMEM((CHUNK,), jnp.int32),
            pltpu.VMEM((CHUNK,), jnp.float32),
            pltpu.VMEM((CHUNK,), jnp.float32),
            pltpu.VMEM((CHUNK, D_MODEL), jnp.float32),
            pltpu.VMEM((CHUNK, D_MODEL), jnp.float32),
            pltpu.VMEM((CHUNK, D_MODEL), jnp.float32),
            pltpu.SemaphoreType.DMA,
            pltpu.SemaphoreType.DMA,
        ],
    )
    def combine(y_hbm, d0_hbm, d1_hbm, g1_hbm, g2_hbm, out_hbm,
                i0_v, i1_v, g1_v, g2_v, r0_v, r1_v, o_v, sem0, sem1):
        wid = lax.axis_index("s") * _SC_INFO.num_cores + lax.axis_index("c")
        base = wid * TOK_PER_WORKER

        def body(c, carry):
            tb = base + c * CHUNK
            pltpu.sync_copy(d0_hbm.at[pl.ds(tb, CHUNK)], i0_v)
            pltpu.sync_copy(d1_hbm.at[pl.ds(tb, CHUNK)], i1_v)
            pltpu.sync_copy(g1_hbm.at[pl.ds(tb, CHUNK)], g1_v)
            pltpu.sync_copy(g2_hbm.at[pl.ds(tb, CHUNK)], g2_v)
            c0 = pltpu.async_copy(y_hbm.at[i0_v], r0_v, sem0)
            c1 = pltpu.async_copy(y_hbm.at[i1_v], r1_v, sem1)
            c0.wait()
            c1.wait()

            gv1 = g1_v[...]
            gv2 = g2_v[...]
            for t in range(CHUNK):
                s1 = gv1[t]
                s2 = gv2[t]

                def dchunk(j, carry3, t=t, s1=s1, s2=s2):
                    sl = pl.ds(j * 16, 16)
                    o_v[t, sl] = r0_v[t, sl] * s1 + r1_v[t, sl] * s2
                    return carry3

                lax.fori_loop(0, D_MODEL // 16, dchunk, 0, unroll=4)
            pltpu.sync_copy(o_v, out_hbm.at[pl.ds(tb, CHUNK)])
            return carry

        lax.fori_loop(0, TOK_PER_WORKER // CHUNK, body, 0)

    return combine


_scatter_fn = _make_scatter()
_combine_fn = _make_combine()


def kernel(x, gate_W, gate_b, expert_W, expert_b):
    e1, e2, g1, g2 = _gating(x, gate_W, gate_b)

    e = jnp.concatenate([e1, e2], axis=1).reshape(-1)
    onehot = e[:, None] == jnp.arange(NUM_EXPERTS, dtype=jnp.int32)[None, :]
    cum = jnp.cumsum(onehot.astype(jnp.int32), axis=0)
    counts = cum[-1]
    padded = (counts + TILE - 1) // TILE * TILE
    off = jnp.concatenate([jnp.zeros((1,), jnp.int32),
                           jnp.cumsum(padded)[:-1].astype(jnp.int32)])
    rank = jnp.sum(jnp.where(onehot, cum - 1, 0), axis=1)
    bases = jnp.sum(jnp.where(onehot, off[None, :], 0), axis=1)
    dest = (bases + rank).astype(jnp.int32)
    dd = dest.reshape(N_TOKENS, TOP_K)
    dest0 = dd[:, 0]
    dest1 = dd[:, 1]
    tile_expert = (jnp.searchsorted(
        off, jnp.arange(NUM_TILES, dtype=jnp.int32) * TILE, side="right")
        .astype(jnp.int32) - 1)
    tile_expert = jnp.clip(tile_expert, 0, NUM_EXPERTS - 1)

    xs = _scatter_fn(x, dest0, dest1)
    y = _gmm(tile_expert, xs, expert_W, expert_b)
    out = _combine_fn(y, dest0, dest1,
                      g1.reshape(-1), g2.reshape(-1))
    return out


# speedup vs baseline: 1.1086x; 1.1086x over previous
import functools

import jax
import jax.numpy as jnp
from jax import lax
from jax.experimental import pallas as pl
from jax.experimental.pallas import tpu as pltpu
from jax.experimental.pallas import tpu_sc as plsc

D_MODEL = 1024
NUM_EXPERTS = 8
TOP_K = 2
N_TOKENS = 4096
N_PAIRS = N_TOKENS * TOP_K

TILE = 256
P_PAD = ((N_PAIRS + NUM_EXPERTS * (TILE - 1)) + TILE - 1) // TILE * TILE
NUM_TILES = P_PAD // TILE

_SC_INFO = plsc.get_sparse_core_info()
NUM_WORKERS = _SC_INFO.num_cores * _SC_INFO.num_subcores
TOK_PER_WORKER = N_TOKENS // NUM_WORKERS
CHUNK = 16


def _gate_kernel(x_ref, gw_ref, gb_ref, e1_ref, e2_ref, g1_ref, g2_ref):
    n = x_ref.shape[0]
    logits = jnp.dot(x_ref[...], gw_ref[...],
                     preferred_element_type=jnp.float32) + gb_ref[...]
    col = lax.broadcasted_iota(jnp.int32, (n, NUM_EXPERTS), 1)
    m1 = jnp.max(logits, axis=1, keepdims=True)
    a1 = jnp.min(jnp.where(logits == m1, col, NUM_EXPERTS), axis=1, keepdims=True)
    neg = jnp.float32(-jnp.inf)
    l2 = jnp.where(col == a1, neg, logits)
    m2 = jnp.max(l2, axis=1, keepdims=True)
    a2 = jnp.min(jnp.where(l2 == m2, col, NUM_EXPERTS), axis=1, keepdims=True)
    g1 = 1.0 / (1.0 + jnp.exp(m2 - m1))
    e1_ref[...] = a1
    e2_ref[...] = a2
    g1_ref[...] = g1
    g2_ref[...] = 1.0 - g1


def _gating(x, gate_W, gate_b):
    gb2 = gate_b.reshape(1, NUM_EXPERTS)
    grid = 16
    blk = N_TOKENS // grid
    outs = [
        jax.ShapeDtypeStruct((N_TOKENS, 1), jnp.int32),
        jax.ShapeDtypeStruct((N_TOKENS, 1), jnp.int32),
        jax.ShapeDtypeStruct((N_TOKENS, 1), jnp.float32),
        jax.ShapeDtypeStruct((N_TOKENS, 1), jnp.float32),
    ]
    ospec = pl.BlockSpec((blk, 1), lambda i: (i, 0))
    return pl.pallas_call(
        _gate_kernel,
        grid=(grid,),
        in_specs=[
            pl.BlockSpec((blk, D_MODEL), lambda i: (i, 0)),
            pl.BlockSpec((D_MODEL, NUM_EXPERTS), lambda i: (0, 0)),
            pl.BlockSpec((1, NUM_EXPERTS), lambda i: (0, 0)),
        ],
        out_specs=[ospec, ospec, ospec, ospec],
        out_shape=outs,
    )(x, gate_W, gb2)


def _make_scatter():
    mesh = plsc.VectorSubcoreMesh(core_axis_name="c", subcore_axis_name="s")

    @functools.partial(
        pl.kernel,
        mesh=mesh,
        out_type=jax.ShapeDtypeStruct((P_PAD, D_MODEL), jnp.float32),
        scratch_types=[
            pltpu.VMEM((CHUNK,), jnp.int32),
            pltpu.VMEM((CHUNK,), jnp.int32),
            pltpu.VMEM((CHUNK, D_MODEL), jnp.float32),
            pltpu.SemaphoreType.DMA,
            pltpu.SemaphoreType.DMA,
        ],
    )
    def scatter(x_hbm, d0_hbm, d1_hbm, xs_hbm, i0_v, i1_v, rows_v, sem0, sem1):
        wid = lax.axis_index("s") * _SC_INFO.num_cores + lax.axis_index("c")
        base = wid * TOK_PER_WORKER

        def body(c, carry):
            tb = base + c * CHUNK
            pltpu.sync_copy(d0_hbm.at[pl.ds(tb, CHUNK)], i0_v)
            pltpu.sync_copy(d1_hbm.at[pl.ds(tb, CHUNK)], i1_v)
            pltpu.sync_copy(x_hbm.at[pl.ds(tb, CHUNK)], rows_v)
            c0 = pltpu.async_copy(rows_v, xs_hbm.at[i0_v], sem0)
            c1 = pltpu.async_copy(rows_v, xs_hbm.at[i1_v], sem1)
            c0.wait()
            c1.wait()
            return carry

        lax.fori_loop(0, TOK_PER_WORKER // CHUNK, body, 0)

    return scatter


def _gmm_kernel(te_ref, xs_ref, w_ref, b_ref, y_ref):
    y_ref[...] = jnp.dot(xs_ref[...].astype(jnp.bfloat16),
                         w_ref[0].astype(jnp.bfloat16),
                         preferred_element_type=jnp.float32) + b_ref[0]


def _gmm(tile_expert, xs, expert_W, expert_b):
    grid_spec = pltpu.PrefetchScalarGridSpec(
        num_scalar_prefetch=1,
        grid=(NUM_TILES,),
        in_specs=[
            pl.BlockSpec((TILE, D_MODEL), lambda i, te: (i, 0)),
            pl.BlockSpec((1, D_MODEL, D_MODEL), lambda i, te: (te[i], 0, 0)),
            pl.BlockSpec((1, 1, D_MODEL), lambda i, te: (te[i], 0, 0)),
        ],
        out_specs=pl.BlockSpec((TILE, D_MODEL), lambda i, te: (i, 0)),
    )
    return pl.pallas_call(
        _gmm_kernel,
        grid_spec=grid_spec,
        out_shape=jax.ShapeDtypeStruct((P_PAD, D_MODEL), jnp.float32),
        compiler_params=pltpu.CompilerParams(
            dimension_semantics=("arbitrary",)),
    )(tile_expert, xs, expert_W,
      expert_b.reshape(NUM_EXPERTS, 1, D_MODEL))


GCHUNK = 32


def _make_gather2():
    mesh = plsc.VectorSubcoreMesh(core_axis_name="c", subcore_axis_name="s")

    @functools.partial(
        pl.kernel,
        mesh=mesh,
        out_type=[
            jax.ShapeDtypeStruct((N_TOKENS, D_MODEL), jnp.float32),
            jax.ShapeDtypeStruct((N_TOKENS, D_MODEL), jnp.float32),
        ],
        scratch_types=[
            pltpu.VMEM((GCHUNK,), jnp.int32),
            pltpu.VMEM((GCHUNK,), jnp.int32),
            pltpu.VMEM((GCHUNK, D_MODEL), jnp.float32),
            pltpu.VMEM((GCHUNK, D_MODEL), jnp.float32),
            pltpu.SemaphoreType.DMA,
            pltpu.SemaphoreType.DMA,
        ],
    )
    def gather2(y_hbm, d0_hbm, d1_hbm, y0_hbm, y1_hbm,
                i0_v, i1_v, r0_v, r1_v, sem0, sem1):
        wid = lax.axis_index("s") * _SC_INFO.num_cores + lax.axis_index("c")
        base = wid * TOK_PER_WORKER

        def body(c, carry):
            tb = base + c * GCHUNK
            pltpu.sync_copy(d0_hbm.at[pl.ds(tb, GCHUNK)], i0_v)
            pltpu.sync_copy(d1_hbm.at[pl.ds(tb, GCHUNK)], i1_v)
            c0 = pltpu.async_copy(y_hbm.at[i0_v], r0_v, sem0)
            c1 = pltpu.async_copy(y_hbm.at[i1_v], r1_v, sem1)
            c0.wait()
            c1.wait()
            pltpu.sync_copy(r0_v, y0_hbm.at[pl.ds(tb, GCHUNK)])
            pltpu.sync_copy(r1_v, y1_hbm.at[pl.ds(tb, GCHUNK)])
            return carry

        lax.fori_loop(0, TOK_PER_WORKER // GCHUNK, body, 0)

    return gather2


def _combine_kernel(y0_ref, y1_ref, g1_ref, g2_ref, o_ref):
    o_ref[...] = y0_ref[...] * g1_ref[...] + y1_ref[...] * g2_ref[...]


def _tc_combine(y0, y1, g1, g2):
    grid = 8
    blk = N_TOKENS // grid
    return pl.pallas_call(
        _combine_kernel,
        grid=(grid,),
        in_specs=[
            pl.BlockSpec((blk, D_MODEL), lambda i: (i, 0)),
            pl.BlockSpec((blk, D_MODEL), lambda i: (i, 0)),
            pl.BlockSpec((blk, 1), lambda i: (i, 0)),
            pl.BlockSpec((blk, 1), lambda i: (i, 0)),
        ],
        out_specs=pl.BlockSpec((blk, D_MODEL), lambda i: (i, 0)),
        out_shape=jax.ShapeDtypeStruct((N_TOKENS, D_MODEL), jnp.float32),
    )(y0, y1, g1, g2)


_scatter_fn = _make_scatter()
_gather2_fn = _make_gather2()


def kernel(x, gate_W, gate_b, expert_W, expert_b):
    e1, e2, g1, g2 = _gating(x, gate_W, gate_b)

    e = jnp.concatenate([e1, e2], axis=1).reshape(-1)
    onehot = e[:, None] == jnp.arange(NUM_EXPERTS, dtype=jnp.int32)[None, :]
    cum = jnp.cumsum(onehot.astype(jnp.int32), axis=0)
    counts = cum[-1]
    padded = (counts + TILE - 1) // TILE * TILE
    off = jnp.concatenate([jnp.zeros((1,), jnp.int32),
                           jnp.cumsum(padded)[:-1].astype(jnp.int32)])
    rank = jnp.sum(jnp.where(onehot, cum - 1, 0), axis=1)
    bases = jnp.sum(jnp.where(onehot, off[None, :], 0), axis=1)
    dest = (bases + rank).astype(jnp.int32)
    dd = dest.reshape(N_TOKENS, TOP_K)
    dest0 = dd[:, 0]
    dest1 = dd[:, 1]
    tile_expert = (jnp.searchsorted(
        off, jnp.arange(NUM_TILES, dtype=jnp.int32) * TILE, side="right")
        .astype(jnp.int32) - 1)
    tile_expert = jnp.clip(tile_expert, 0, NUM_EXPERTS - 1)

    xs = _scatter_fn(x, dest0, dest1)
    y = _gmm(tile_expert, xs, expert_W, expert_b)
    y0, y1 = _gather2_fn(y, dest0, dest1)
    return _tc_combine(y0, y1, g1, g2)
